# baseline (speedup 1.0000x reference)
import functools

import jax
import jax.numpy as jnp
from jax import lax
from jax.experimental import pallas as pl
from jax.experimental.pallas import tpu as pltpu
from jax.experimental.pallas import tpu_sc as plsc

D_MODEL = 2048
MAXLEN = 4096
B = 4 * 4096
NC, NS, L = 2, 16, 16
NW = NC * NS
B_PER_W = B // NW
C = 8
NBUF = 4
NCHUNK = B_PER_W // C

G = 16


def _clamp(v):
    return jnp.minimum(jnp.maximum(v, -MAXLEN), MAXLEN - 1) + MAXLEN


def _sc_body(idx_hbm, tbl_hbm, out_hbm, idx_v, *bufsems):
    bufs = bufsems[:NBUF]
    gsems = bufsems[NBUF:2 * NBUF]
    wsems = bufsems[2 * NBUF:3 * NBUF]

    wid = lax.axis_index("s") * NC + lax.axis_index("c")
    base = wid * B_PER_W
    pltpu.sync_copy(idx_hbm.at[pl.ds(base, B_PER_W)], idx_v)

    @pl.loop(0, B_PER_W // L)
    def _(i):
        s = pl.ds(i * L, L)
        idx_v[s] = _clamp(idx_v[s])

    def gather(j, b):
        pltpu.async_copy(
            tbl_hbm.at[idx_v.at[pl.ds(j * C, C)]], bufs[b], gsems[b])

    for b in range(NBUF):
        gather(b, b)

    @pl.loop(0, NCHUNK, step=NBUF)
    def _(j):
        for b in range(NBUF):
            jj = j + b
            pltpu.make_async_copy(
                tbl_hbm.at[idx_v.at[pl.ds(jj * C, C)]], bufs[b],
                gsems[b]).wait()
            pltpu.async_copy(
                bufs[b], out_hbm.at[pl.ds(base + jj * C, C)], wsems[b])
        for b in range(NBUF):
            jj = j + b + NBUF

            @pl.when(jj < NCHUNK)
            def _():
                pltpu.make_async_copy(
                    bufs[b], out_hbm.at[pl.ds(base + (jj - NBUF) * C, C)],
                    wsems[b]).wait()
                gather(jj, b)

    for b in range(NBUF):
        jj = NCHUNK - NBUF + b
        pltpu.make_async_copy(
            bufs[b], out_hbm.at[pl.ds(base + jj * C, C)], wsems[b]).wait()


def _sc_gather(idx_flat, table):
    mesh = plsc.VectorSubcoreMesh(core_axis_name="c", subcore_axis_name="s")
    f = pl.kernel(
        _sc_body,
        mesh=mesh,
        out_type=jax.ShapeDtypeStruct((B, D_MODEL), jnp.float32),
        scratch_types=(
            [pltpu.VMEM((B_PER_W,), jnp.int32)]
            + [pltpu.VMEM((C, D_MODEL), jnp.float32) for _ in range(NBUF)]
            + [pltpu.SemaphoreType.DMA for _ in range(2 * NBUF)]
        ),
    )
    return f(idx_flat, table)


def _tc_body(idx_ref, *refs):
    out = refs[G]
    for t in range(G):
        out[t, :] = refs[t][0, 0, :]


def _tc_gather(idx_flat, table):
    table3 = table.reshape(table.shape[0], 1, D_MODEL)
    in_specs = [
        pl.BlockSpec(
            (1, 1, D_MODEL),
            (lambda i, idx_ref, t=t: (_clamp(idx_ref[G * i + t]), 0, 0)))
        for t in range(G)
    ]
    out_spec = pl.BlockSpec((G, D_MODEL), lambda i, idx_ref: (i, 0))
    return pl.pallas_call(
        _tc_body,
        grid_spec=pltpu.PrefetchScalarGridSpec(
            num_scalar_prefetch=1,
            grid=(B // G,),
            in_specs=in_specs,
            out_specs=out_spec,
        ),
        out_shape=jax.ShapeDtypeStruct((B, D_MODEL), jnp.float32),
        compiler_params=pltpu.CompilerParams(
            dimension_semantics=("parallel",)),
    )(idx_flat, *([table3] * G))


@jax.jit
def _run(idx_flat, pe_k, pe_v):
    return _sc_gather(idx_flat, pe_k), _tc_gather(idx_flat, pe_v)


def kernel(pos_seq, pe_k, pe_v):
    lead = pos_seq.shape
    idx_flat = pos_seq.reshape(B)
    ok, ov = _run(idx_flat, pe_k, pe_v)
    return (ok.reshape(*lead, D_MODEL), ov.reshape(*lead, D_MODEL))


# speedup vs baseline: 1.0857x; 1.0857x over previous
import functools

import jax
import jax.numpy as jnp
from jax import lax
from jax.experimental import pallas as pl
from jax.experimental.pallas import tpu as pltpu
from jax.experimental.pallas import tpu_sc as plsc

D_MODEL = 2048
MAXLEN = 4096
B = 4 * 4096
NC, NS, L = 2, 16, 16
NW = NC * NS
B_PER_W = B // NW
C = 32
NCHUNK = B_PER_W // C

G = 32


def _clamp(v):
    return jnp.minimum(jnp.maximum(v, -MAXLEN), MAXLEN - 1) + MAXLEN


def _sc_body(idx_hbm, tbl_hbm, out_hbm, idx_v, spbuf, gsem):
    sid = lax.axis_index("s")
    wid = sid * NC + lax.axis_index("c")
    base = wid * B_PER_W
    pltpu.sync_copy(idx_hbm.at[pl.ds(base, B_PER_W)], idx_v)

    @pl.loop(0, B_PER_W // L)
    def _(i):
        s = pl.ds(i * L, L)
        idx_v[s] = _clamp(idx_v[s])

    @pl.loop(0, NCHUNK)
    def _(j):
        pltpu.async_copy(
            tbl_hbm.at[idx_v.at[pl.ds(j * C, C)]], spbuf, gsem).wait()
        pltpu.sync_copy(spbuf, out_hbm.at[pl.ds(base + j * C, C)])


def _sc_gather(idx_flat, table):
    mesh = plsc.VectorSubcoreMesh(core_axis_name="c", subcore_axis_name="s")
    f = pl.kernel(
        _sc_body,
        mesh=mesh,
        out_type=jax.ShapeDtypeStruct((B, D_MODEL), jnp.float32),
        scratch_types=[
            pltpu.VMEM((B_PER_W,), jnp.int32),
            pltpu.VMEM((C, D_MODEL), jnp.float32),
            pltpu.SemaphoreType.DMA,
        ],
    )
    return f(idx_flat, table)


def _tc_body(idx_ref, *refs):
    out = refs[G]
    for t in range(G):
        out[t, :] = refs[t][0, 0, :]


def _tc_gather(idx_flat, table):
    table3 = table.reshape(table.shape[0], 1, D_MODEL)
    in_specs = [
        pl.BlockSpec(
            (1, 1, D_MODEL),
            (lambda i, idx_ref, t=t: (_clamp(idx_ref[G * i + t]), 0, 0)))
        for t in range(G)
    ]
    out_spec = pl.BlockSpec((G, D_MODEL), lambda i, idx_ref: (i, 0))
    return pl.pallas_call(
        _tc_body,
        grid_spec=pltpu.PrefetchScalarGridSpec(
            num_scalar_prefetch=1,
            grid=(B // G,),
            in_specs=in_specs,
            out_specs=out_spec,
        ),
        out_shape=jax.ShapeDtypeStruct((B, D_MODEL), jnp.float32),
        compiler_params=pltpu.CompilerParams(
            dimension_semantics=("parallel",)),
    )(idx_flat, *([table3] * G))


@jax.jit
def _run(idx_flat, pe_k, pe_v):
    return _sc_gather(idx_flat, pe_k), _tc_gather(idx_flat, pe_v)


def kernel(pos_seq, pe_k, pe_v):
    lead = pos_seq.shape
    idx_flat = pos_seq.reshape(B)
    ok, ov = _run(idx_flat, pe_k, pe_v)
    return (ok.reshape(*lead, D_MODEL), ov.reshape(*lead, D_MODEL))


# speedup vs baseline: 1.1052x; 1.0179x over previous
import functools

import jax
import jax.numpy as jnp
from jax import lax
from jax.experimental import pallas as pl
from jax.experimental.pallas import tpu as pltpu
from jax.experimental.pallas import tpu_sc as plsc

D_MODEL = 2048
MAXLEN = 4096
B = 4 * 4096
NC, NS, L = 2, 16, 16
NW = NC * NS
B_PER_W = B // NW
C = 32
NCHUNK = B_PER_W // C

G = 64


def _clamp(v):
    return jnp.minimum(jnp.maximum(v, -MAXLEN), MAXLEN - 1) + MAXLEN


def _sc_body(idx_hbm, tbl_hbm, out_hbm, idx_v, spbuf, gsem):
    sid = lax.axis_index("s")
    wid = sid * NC + lax.axis_index("c")
    base = wid * B_PER_W
    pltpu.sync_copy(idx_hbm.at[pl.ds(base, B_PER_W)], idx_v)

    @pl.loop(0, B_PER_W // L)
    def _(i):
        s = pl.ds(i * L, L)
        idx_v[s] = _clamp(idx_v[s])

    @pl.loop(0, NCHUNK)
    def _(j):
        pltpu.async_copy(
            tbl_hbm.at[idx_v.at[pl.ds(j * C, C)]], spbuf, gsem).wait()
        pltpu.sync_copy(spbuf, out_hbm.at[pl.ds(base + j * C, C)])


def _sc_gather(idx_flat, table):
    mesh = plsc.VectorSubcoreMesh(core_axis_name="c", subcore_axis_name="s")
    f = pl.kernel(
        _sc_body,
        mesh=mesh,
        out_type=jax.ShapeDtypeStruct((B, D_MODEL), jnp.float32),
        scratch_types=[
            pltpu.VMEM((B_PER_W,), jnp.int32),
            pltpu.VMEM((C, D_MODEL), jnp.float32),
            pltpu.SemaphoreType.DMA,
        ],
    )
    return f(idx_flat, table)


def _tc_body(idx_ref, *refs):
    out = refs[G]
    for t in range(G):
        out[t, :] = refs[t][0, 0, :]


def _tc_gather(idx_flat, table):
    table3 = table.reshape(table.shape[0], 1, D_MODEL)
    in_specs = [
        pl.BlockSpec(
            (1, 1, D_MODEL),
            (lambda i, idx_ref, t=t: (_clamp(idx_ref[G * i + t]), 0, 0)))
        for t in range(G)
    ]
    out_spec = pl.BlockSpec((G, D_MODEL), lambda i, idx_ref: (i, 0))
    return pl.pallas_call(
        _tc_body,
        grid_spec=pltpu.PrefetchScalarGridSpec(
            num_scalar_prefetch=1,
            grid=(B // G,),
            in_specs=in_specs,
            out_specs=out_spec,
        ),
        out_shape=jax.ShapeDtypeStruct((B, D_MODEL), jnp.float32),
        compiler_params=pltpu.CompilerParams(
            dimension_semantics=("parallel",)),
    )(idx_flat, *([table3] * G))


@jax.jit
def _run(idx_flat, pe_k, pe_v):
    return _sc_gather(idx_flat, pe_k), _tc_gather(idx_flat, pe_v)


def kernel(pos_seq, pe_k, pe_v):
    lead = pos_seq.shape
    idx_flat = pos_seq.reshape(B)
    ok, ov = _run(idx_flat, pe_k, pe_v)
    return (ok.reshape(*lead, D_MODEL), ov.reshape(*lead, D_MODEL))


# speedup vs baseline: 1.1167x; 1.0104x over previous
import functools

import jax
import jax.numpy as jnp
from jax import lax
from jax.experimental import pallas as pl
from jax.experimental.pallas import tpu as pltpu
from jax.experimental.pallas import tpu_sc as plsc

D_MODEL = 2048
MAXLEN = 4096
B = 4 * 4096
NC, NS, L = 2, 16, 16
NW = NC * NS
B_PER_W = B // NW
C = 32
NCHUNK = B_PER_W // C

G = 128


def _clamp(v):
    return jnp.minimum(jnp.maximum(v, -MAXLEN), MAXLEN - 1) + MAXLEN


def _sc_body(idx_hbm, tbl_hbm, out_hbm, idx_v, spbuf, gsem):
    sid = lax.axis_index("s")
    wid = sid * NC + lax.axis_index("c")
    base = wid * B_PER_W
    pltpu.sync_copy(idx_hbm.at[pl.ds(base, B_PER_W)], idx_v)

    @pl.loop(0, B_PER_W // L)
    def _(i):
        s = pl.ds(i * L, L)
        idx_v[s] = _clamp(idx_v[s])

    @pl.loop(0, NCHUNK)
    def _(j):
        pltpu.async_copy(
            tbl_hbm.at[idx_v.at[pl.ds(j * C, C)]], spbuf, gsem).wait()
        pltpu.sync_copy(spbuf, out_hbm.at[pl.ds(base + j * C, C)])


def _sc_gather(idx_flat, table):
    mesh = plsc.VectorSubcoreMesh(core_axis_name="c", subcore_axis_name="s")
    f = pl.kernel(
        _sc_body,
        mesh=mesh,
        out_type=jax.ShapeDtypeStruct((B, D_MODEL), jnp.float32),
        scratch_types=[
            pltpu.VMEM((B_PER_W,), jnp.int32),
            pltpu.VMEM((C, D_MODEL), jnp.float32),
            pltpu.SemaphoreType.DMA,
        ],
    )
    return f(idx_flat, table)


def _tc_body(idx_ref, *refs):
    out = refs[G]
    for t in range(G):
        out[t, :] = refs[t][0, 0, :]


def _tc_gather(idx_flat, table):
    table3 = table.reshape(table.shape[0], 1, D_MODEL)
    in_specs = [
        pl.BlockSpec(
            (1, 1, D_MODEL),
            (lambda i, idx_ref, t=t: (_clamp(idx_ref[G * i + t]), 0, 0)))
        for t in range(G)
    ]
    out_spec = pl.BlockSpec((G, D_MODEL), lambda i, idx_ref: (i, 0))
    return pl.pallas_call(
        _tc_body,
        grid_spec=pltpu.PrefetchScalarGridSpec(
            num_scalar_prefetch=1,
            grid=(B // G,),
            in_specs=in_specs,
            out_specs=out_spec,
        ),
        out_shape=jax.ShapeDtypeStruct((B, D_MODEL), jnp.float32),
        compiler_params=pltpu.CompilerParams(
            dimension_semantics=("parallel",)),
    )(idx_flat, *([table3] * G))


@jax.jit
def _run(idx_flat, pe_k, pe_v):
    return _sc_gather(idx_flat, pe_k), _tc_gather(idx_flat, pe_v)


def kernel(pos_seq, pe_k, pe_v):
    lead = pos_seq.shape
    idx_flat = pos_seq.reshape(B)
    ok, ov = _run(idx_flat, pe_k, pe_v)
    return (ok.reshape(*lead, D_MODEL), ov.reshape(*lead, D_MODEL))


# speedup vs baseline: 1.1233x; 1.0059x over previous
import functools

import jax
import jax.numpy as jnp
from jax import lax
from jax.experimental import pallas as pl
from jax.experimental.pallas import tpu as pltpu
from jax.experimental.pallas import tpu_sc as plsc

D_MODEL = 2048
MAXLEN = 4096
B = 4 * 4096
NC, NS, L = 2, 16, 16
NW = NC * NS
B_PER_W = B // NW
C = 32
NCHUNK = B_PER_W // C

G = 256


def _clamp(v):
    return jnp.minimum(jnp.maximum(v, -MAXLEN), MAXLEN - 1) + MAXLEN


def _sc_body(idx_hbm, tbl_hbm, out_hbm, idx_v, spbuf, gsem):
    sid = lax.axis_index("s")
    wid = sid * NC + lax.axis_index("c")
    base = wid * B_PER_W
    pltpu.sync_copy(idx_hbm.at[pl.ds(base, B_PER_W)], idx_v)

    @pl.loop(0, B_PER_W // L)
    def _(i):
        s = pl.ds(i * L, L)
        idx_v[s] = _clamp(idx_v[s])

    @pl.loop(0, NCHUNK)
    def _(j):
        pltpu.async_copy(
            tbl_hbm.at[idx_v.at[pl.ds(j * C, C)]], spbuf, gsem).wait()
        pltpu.sync_copy(spbuf, out_hbm.at[pl.ds(base + j * C, C)])


def _sc_gather(idx_flat, table):
    mesh = plsc.VectorSubcoreMesh(core_axis_name="c", subcore_axis_name="s")
    f = pl.kernel(
        _sc_body,
        mesh=mesh,
        out_type=jax.ShapeDtypeStruct((B, D_MODEL), jnp.float32),
        scratch_types=[
            pltpu.VMEM((B_PER_W,), jnp.int32),
            pltpu.VMEM((C, D_MODEL), jnp.float32),
            pltpu.SemaphoreType.DMA,
        ],
    )
    return f(idx_flat, table)


def _tc_body(idx_ref, *refs):
    out = refs[G]
    for t in range(G):
        out[t, :] = refs[t][0, 0, :]


def _tc_gather(idx_flat, table):
    table3 = table.reshape(table.shape[0], 1, D_MODEL)
    in_specs = [
        pl.BlockSpec(
            (1, 1, D_MODEL),
            (lambda i, idx_ref, t=t: (_clamp(idx_ref[G * i + t]), 0, 0)))
        for t in range(G)
    ]
    out_spec = pl.BlockSpec((G, D_MODEL), lambda i, idx_ref: (i, 0))
    return pl.pallas_call(
        _tc_body,
        grid_spec=pltpu.PrefetchScalarGridSpec(
            num_scalar_prefetch=1,
            grid=(B // G,),
            in_specs=in_specs,
            out_specs=out_spec,
        ),
        out_shape=jax.ShapeDtypeStruct((B, D_MODEL), jnp.float32),
        compiler_params=pltpu.CompilerParams(
            dimension_semantics=("parallel",)),
    )(idx_flat, *([table3] * G))


@jax.jit
def _run(idx_flat, pe_k, pe_v):
    return _sc_gather(idx_flat, pe_k), _tc_gather(idx_flat, pe_v)


def kernel(pos_seq, pe_k, pe_v):
    lead = pos_seq.shape
    idx_flat = pos_seq.reshape(B)
    ok, ov = _run(idx_flat, pe_k, pe_v)
    return (ok.reshape(*lead, D_MODEL), ov.reshape(*lead, D_MODEL))


# speedup vs baseline: 1.1244x; 1.0010x over previous
import jax
import jax.numpy as jnp
from jax import lax
from jax.experimental import pallas as pl
from jax.experimental.pallas import tpu as pltpu
from jax.experimental.pallas import tpu_sc as plsc

D_MODEL = 2048
MAXLEN = 4096
B = 4 * 4096
NC, NS, L = 2, 16, 16
NW = NC * NS
B_PER_W = B // NW
C = 32
NCHUNK = B_PER_W // C

G = 256


def _clamp(v):
    return jnp.minimum(jnp.maximum(v, -MAXLEN), MAXLEN - 1) + MAXLEN


def _sc_body(idx_hbm, tbl_hbm, out_hbm, idx_v, spbuf, gsem):
    sid = lax.axis_index("s")
    wid = sid * NC + lax.axis_index("c")
    base = wid * B_PER_W
    pltpu.sync_copy(idx_hbm.at[pl.ds(base, B_PER_W)], idx_v)

    @pl.loop(0, B_PER_W // L)
    def _(i):
        s = pl.ds(i * L, L)
        idx_v[s] = _clamp(idx_v[s])

    @pl.loop(0, NCHUNK)
    def _(j):
        pltpu.async_copy(
            tbl_hbm.at[idx_v.at[pl.ds(j * C, C)]], spbuf, gsem).wait()
        pltpu.sync_copy(spbuf, out_hbm.at[pl.ds(base + j * C, C)])


def _sc_gather(idx_flat, table):
    mesh = plsc.VectorSubcoreMesh(core_axis_name="c", subcore_axis_name="s")
    f = pl.kernel(
        _sc_body,
        mesh=mesh,
        out_type=jax.ShapeDtypeStruct((B, D_MODEL), jnp.float32),
        scratch_types=[
            pltpu.VMEM((B_PER_W,), jnp.int32),
            pltpu.VMEM((C, D_MODEL), jnp.float32),
            pltpu.SemaphoreType.DMA,
        ],
    )
    return f(idx_flat, table)


def _tc_body(idx_ref, *refs):
    out = refs[G]
    for t in range(G):
        out[t, :] = refs[t][0, 0, :]


def _tc_gather(idx_flat, table):
    table3 = table.reshape(table.shape[0], 1, D_MODEL)
    in_specs = [
        pl.BlockSpec(
            (1, 1, D_MODEL),
            (lambda i, idx_ref, t=t: (_clamp(idx_ref[G * i + t]), 0, 0)))
        for t in range(G)
    ]
    out_spec = pl.BlockSpec((G, D_MODEL), lambda i, idx_ref: (i, 0))
    return pl.pallas_call(
        _tc_body,
        grid_spec=pltpu.PrefetchScalarGridSpec(
            num_scalar_prefetch=1,
            grid=(B // G,),
            in_specs=in_specs,
            out_specs=out_spec,
        ),
        out_shape=jax.ShapeDtypeStruct((B, D_MODEL), jnp.float32),
        compiler_params=pltpu.CompilerParams(
            dimension_semantics=("parallel",)),
    )(idx_flat, *([table3] * G))


@jax.jit
def _run(idx_flat, pe_k, pe_v):
    return _sc_gather(idx_flat, pe_k), _tc_gather(idx_flat, pe_v)


def kernel(pos_seq, pe_k, pe_v):
    lead = pos_seq.shape
    idx_flat = pos_seq.reshape(B)
    ok, ov = _run(idx_flat, pe_k, pe_v)
    return (ok.reshape(*lead, D_MODEL), ov.reshape(*lead, D_MODEL))
